# baseline (speedup 1.0000x reference)
import functools
import math

import jax
import jax.numpy as jnp
from jax import lax
from jax.experimental import pallas as pl
from jax.experimental.pallas import tpu as pltpu
from jax.experimental.pallas import tpu_sc as plsc

EMBED_DIM = 64
SCALE = math.sqrt(EMBED_DIM)

NUM_CORES = 2
NUM_SUBCORES = 16
NUM_WORKERS = NUM_CORES * NUM_SUBCORES
LANES = 16

CHUNK = 512


def _make_kernel(batch: int):
    assert batch % (8 * NUM_WORKERS) == 0
    rows_per_worker = batch // NUM_WORKERS
    assert rows_per_worker % CHUNK == 0
    n_chunks = rows_per_worker // CHUNK

    mesh = plsc.VectorSubcoreMesh(
        core_axis_name="c", subcore_axis_name="s"
    )

    @functools.partial(
        pl.kernel,
        mesh=mesh,
        compiler_params=pltpu.CompilerParams(use_tc_tiling_on_sc=False),
        out_type=jax.ShapeDtypeStruct((batch, EMBED_DIM), jnp.float32),
        scratch_types=[
            pltpu.VMEM((CHUNK,), jnp.int32),
            pltpu.VMEM((CHUNK, EMBED_DIM), jnp.float32),
            pltpu.SemaphoreType.DMA,
        ],
    )
    def emb_kernel(x_hbm, table_hbm, out_hbm, idx_v, rows_v, sem):
        wid = lax.axis_index("s") * NUM_CORES + lax.axis_index("c")
        base = wid * rows_per_worker

        def chunk_body(i, carry):
            off = base + i * CHUNK
            pltpu.sync_copy(x_hbm.at[pl.ds(off, CHUNK)], idx_v)
            pltpu.async_copy(table_hbm.at[idx_v], rows_v, sem).wait()

            def mul_row(r, c):
                for j in range(EMBED_DIM // LANES):
                    sl = rows_v[r, pl.ds(j * LANES, LANES)]
                    rows_v[r, pl.ds(j * LANES, LANES)] = sl * SCALE
                return c

            lax.fori_loop(0, CHUNK, mul_row, 0)
            pltpu.sync_copy(rows_v, out_hbm.at[pl.ds(off, CHUNK)])
            return carry

        lax.fori_loop(0, n_chunks, chunk_body, 0)

    return emb_kernel


def kernel(x, table):
    b, h = x.shape
    batch = b * h
    out = _make_kernel(batch)(x.reshape(batch), table)
    return out.reshape(b, h, EMBED_DIM)


# speedup vs baseline: 1.1302x; 1.1302x over previous
import functools
import math

import jax
import jax.numpy as jnp
from jax import lax
from jax.experimental import pallas as pl
from jax.experimental.pallas import tpu as pltpu
from jax.experimental.pallas import tpu_sc as plsc

EMBED_DIM = 64
SCALE = math.sqrt(EMBED_DIM)

NUM_CORES = 2
NUM_SUBCORES = 16
NUM_WORKERS = NUM_CORES * NUM_SUBCORES
LANES = 16

CHUNK = 800
ROWS_PER_ITER = 4


def _make_kernel(batch: int):
    assert batch % (8 * NUM_WORKERS) == 0
    rows_per_worker = batch // NUM_WORKERS
    assert rows_per_worker % CHUNK == 0
    n_chunks = rows_per_worker // CHUNK

    mesh = plsc.VectorSubcoreMesh(
        core_axis_name="c", subcore_axis_name="s"
    )

    @functools.partial(
        pl.kernel,
        mesh=mesh,
        compiler_params=pltpu.CompilerParams(use_tc_tiling_on_sc=False),
        out_type=jax.ShapeDtypeStruct((batch, EMBED_DIM), jnp.float32),
        scratch_types=[
            pltpu.VMEM((CHUNK,), jnp.int32),
            pltpu.VMEM((CHUNK,), jnp.int32),
            pltpu.VMEM((CHUNK, EMBED_DIM), jnp.float32),
            pltpu.VMEM((CHUNK, EMBED_DIM), jnp.float32),
            pltpu.SemaphoreType.DMA,
            pltpu.SemaphoreType.DMA,
            pltpu.SemaphoreType.DMA,
            pltpu.SemaphoreType.DMA,
        ],
    )
    def emb_kernel(x_hbm, table_hbm, out_hbm, idx0, idx1, rows0, rows1,
                   sg0, sg1, ss0, ss1):
        wid = lax.axis_index("s") * NUM_CORES + lax.axis_index("c")
        base = wid * rows_per_worker

        idx_v = (idx0, idx1)
        rows_v = (rows0, rows1)
        sg = (sg0, sg1)
        ss = (ss0, ss1)

        def start_gather(chunk_i, b):
            off = base + chunk_i * CHUNK
            pltpu.sync_copy(x_hbm.at[pl.ds(off, CHUNK)], idx_v[b])
            return pltpu.async_copy(table_hbm.at[idx_v[b]], rows_v[b], sg[b])

        def scale_rows(b):
            rv = rows_v[b]

            def body(r0, c):
                r = r0 * ROWS_PER_ITER
                for dr in range(ROWS_PER_ITER):
                    for j in range(EMBED_DIM // LANES):
                        sl = rv[r + dr, pl.ds(j * LANES, LANES)]
                        rv[r + dr, pl.ds(j * LANES, LANES)] = sl * SCALE
                return c

            lax.fori_loop(0, CHUNK // ROWS_PER_ITER, body, 0)

        gather_h = [None, None]
        store_h = [None, None]

        gather_h[0] = start_gather(0, 0)
        for i in range(n_chunks):
            b = i % 2
            if i + 1 < n_chunks:
                nb = (i + 1) % 2
                if store_h[nb] is not None:
                    store_h[nb].wait()
                    store_h[nb] = None
                gather_h[nb] = start_gather(i + 1, nb)
            gather_h[b].wait()
            scale_rows(b)
            off = base + i * CHUNK
            store_h[b] = pltpu.async_copy(
                rows_v[b], out_hbm.at[pl.ds(off, CHUNK)], ss[b]
            )
        for b in range(2):
            if store_h[b] is not None:
                store_h[b].wait()

    return emb_kernel


def kernel(x, table):
    b, h = x.shape
    batch = b * h
    out = _make_kernel(batch)(x.reshape(batch), table)
    return out.reshape(b, h, EMBED_DIM)
